# baseline (speedup 1.0000x reference)
import functools

import jax
import jax.numpy as jnp
from jax import lax
from jax.experimental import pallas as pl
from jax.experimental.pallas import tpu as pltpu
from jax.experimental.pallas import tpu_sc as plsc

_LANES = 16


def kernel(labels, table):
    (B,) = labels.shape
    V, D = table.shape
    info = plsc.get_sparse_core_info()
    nw = info.num_cores * info.num_subcores
    b_per_w = B // nw
    n_groups = b_per_w // _LANES
    pack = 128 // D

    table2 = table.reshape(V // pack, 128)

    mesh = plsc.VectorSubcoreMesh(core_axis_name="c", subcore_axis_name="s")

    @functools.partial(
        pl.kernel,
        mesh=mesh,
        out_type=jax.ShapeDtypeStruct((B, D), jnp.float32),
        scratch_types=[
            pltpu.VMEM((b_per_w,), jnp.int32),
            pltpu.VMEM((b_per_w,), jnp.int32),
            pltpu.VMEM((b_per_w // 2, 128), jnp.float32),
            pltpu.VMEM((b_per_w, D), jnp.float32),
            pltpu.SemaphoreType.DMA,
        ],
        compiler_params=pltpu.CompilerParams(needs_layout_passes=False),
    )
    def gather_kernel(labels_hbm, table_hbm, out_hbm, idx_v, sup_v, rows_v,
                      out_v, sem):
        wid = lax.axis_index("s") * info.num_cores + lax.axis_index("c")
        base = wid * b_per_w
        pltpu.sync_copy(labels_hbm.at[pl.ds(base, b_per_w)], idx_v)

        def compute_super(g, _):
            v = idx_v[pl.ds(g * _LANES, _LANES)]
            sup_v[pl.ds(g * _LANES, _LANES)] = lax.shift_right_logical(v, 2)
            return _

        lax.fori_loop(0, n_groups, compute_super, 0, unroll=4)

        c_rows = b_per_w // 2
        for h in range(2):
            off = h * c_rows
            pltpu.async_copy(
                table_hbm.at[sup_v.at[pl.ds(off, c_rows)]], rows_v, sem
            ).wait()

            def extract(g, _):
                row_vec = g * _LANES + lax.iota(jnp.int32, _LANES)
                lab = idx_v[pl.ds(off + g * _LANES, _LANES)]
                colbase = (lab & (pack - 1)) * D
                for c in range(D):
                    vals = plsc.load_gather(rows_v, [row_vec, colbase + c])
                    plsc.store_scatter(
                        out_v,
                        [off + row_vec, jnp.full((_LANES,), c, jnp.int32)],
                        vals)
                return _

            lax.fori_loop(0, c_rows // _LANES, extract, 0)

        pltpu.sync_copy(out_v, out_hbm.at[pl.ds(base, b_per_w)])

    return gather_kernel(labels.astype(jnp.int32), table2)


# speedup vs baseline: 1.2365x; 1.2365x over previous
import functools

import jax
import jax.numpy as jnp
from jax import lax
from jax.experimental import pallas as pl
from jax.experimental.pallas import tpu as pltpu
from jax.experimental.pallas import tpu_sc as plsc


def kernel(labels, table):
    (B,) = labels.shape
    V, D = table.shape
    info = plsc.get_sparse_core_info()
    nw = info.num_cores * info.num_subcores
    b_per_w = B // nw

    table3 = table.reshape(V // 8, 8, D)

    mesh = plsc.VectorSubcoreMesh(core_axis_name="c", subcore_axis_name="s")

    @functools.partial(
        pl.kernel,
        mesh=mesh,
        out_type=jax.ShapeDtypeStruct((B, D), jnp.float32),
        scratch_types=[
            pltpu.VMEM((b_per_w,), jnp.int32),
            pltpu.SemaphoreType.DMA,
        ],
    )
    def gather_kernel(labels_hbm, table_hbm, out_hbm, idx_v, sem):
        wid = lax.axis_index("s") * info.num_cores + lax.axis_index("c")
        base = wid * b_per_w
        pltpu.sync_copy(labels_hbm.at[pl.ds(base, b_per_w)], idx_v)

        def fire_group(g, _):
            v = idx_v[pl.ds(g * 16, 16)]
            j = base + g * 16
            for k in range(16):
                l = v[k]
                pltpu.make_async_copy(
                    table_hbm.at[lax.shift_right_logical(l, 3), l & 7],
                    out_hbm.at[j + k],
                    sem,
                ).start()
            return _

        lax.fori_loop(0, b_per_w // 16, fire_group, 0)

        pltpu.make_async_copy(
            out_hbm.at[pl.ds(base, b_per_w)],
            out_hbm.at[pl.ds(base, b_per_w)],
            sem,
        ).wait()

    return gather_kernel(labels.astype(jnp.int32), table3)


# speedup vs baseline: 2.8495x; 2.3045x over previous
import functools

import jax
import jax.numpy as jnp
from jax import lax
from jax.experimental import pallas as pl
from jax.experimental.pallas import tpu as pltpu
from jax.experimental.pallas import tpu_sc as plsc


def kernel(labels, table):
    (B,) = labels.shape
    V, D = table.shape
    info = plsc.get_sparse_core_info()
    nw = info.num_cores * info.num_subcores
    b_per_w = B // nw

    table3 = table.reshape(V // 8, 8, D)

    mesh = plsc.VectorSubcoreMesh(core_axis_name="c", subcore_axis_name="s")

    @functools.partial(
        pl.kernel,
        mesh=mesh,
        out_type=jax.ShapeDtypeStruct((B, D), jnp.float32),
        scratch_types=[
            pltpu.VMEM((b_per_w,), jnp.int32),
            pltpu.VMEM((b_per_w, D), jnp.float32),
            pltpu.SemaphoreType.DMA,
        ],
    )
    def gather_kernel(labels_hbm, table_hbm, out_hbm, idx_v, out_v, sem):
        wid = lax.axis_index("s") * info.num_cores + lax.axis_index("c")
        base = wid * b_per_w
        pltpu.sync_copy(labels_hbm.at[pl.ds(base, b_per_w)], idx_v)

        def fire_group(g, _):
            v = idx_v[pl.ds(g * 16, 16)]
            for k in range(16):
                l = v[k]
                pltpu.make_async_copy(
                    table_hbm.at[lax.shift_right_logical(l, 3), l & 7],
                    out_v.at[g * 16 + k],
                    sem,
                ).start()
            return _

        lax.fori_loop(0, b_per_w // 16, fire_group, 0)

        pltpu.make_async_copy(
            out_hbm.at[pl.ds(base, b_per_w)],
            out_v,
            sem,
        ).wait()

        pltpu.sync_copy(out_v, out_hbm.at[pl.ds(base, b_per_w)])

    return gather_kernel(labels.astype(jnp.int32), table3)
